# baseline (speedup 1.0000x reference)
import functools

import jax
import jax.numpy as jnp
from jax import lax
from jax.experimental import pallas as pl
from jax.experimental.pallas import tpu as pltpu
from jax.experimental.pallas import tpu_sc as plsc

N = 10000
E = 160000
B = 64
DIM = 32
HID = 128
DD = DIM * DIM

NC = 2
NS = 16
NW = NC * NS
CHUNK = 128
NPAD = 10016
EPAD = 163840
CPW = EPAD // (NW * CHUNK)
RPT = NPAD // NS
CNTW = 16
GRP = 8


@functools.cache
def _sc_kernels():
    mesh = plsc.VectorSubcoreMesh(core_axis_name="c", subcore_axis_name="s",
                                  num_cores=NC, num_subcores=NS)
    params = pltpu.CompilerParams(use_tc_tiling_on_sc=False)

    @functools.partial(
        pl.kernel,
        out_type=jax.ShapeDtypeStruct((EPAD, 128), jnp.bfloat16),
        mesh=mesh,
        scratch_types=[
            pltpu.VMEM((CPW, CHUNK), jnp.int32),
            pltpu.VMEM((GRP, CHUNK, DIM), jnp.bfloat16),
            pltpu.SemaphoreType.DMA,
            pltpu.SemaphoreType.DMA,
        ],
        compiler_params=params,
    )
    def sc_gather(table, idx, out, idx_v, rows_v, sem, sem2):
        w = lax.axis_index("s") * NC + lax.axis_index("c")
        base = w * CPW
        pltpu.sync_copy(idx.at[pl.ds(base, CPW)], idx_v)

        def body(g, carry):
            j0 = g * GRP
            ds = [pltpu.async_copy(table.at[idx_v.at[j0 + k]], rows_v.at[k], sem)
                  for k in range(GRP)]
            for d in ds:
                d.wait()
            ds = [pltpu.async_copy(
                rows_v.at[k],
                out.at[pl.ds((base + j0 + k) * CHUNK, CHUNK), pl.ds(0, DIM)],
                sem2) for k in range(GRP)]
            for d in ds:
                d.wait()
            return carry

        lax.fori_loop(0, CPW // GRP, body, 0)

    @functools.partial(
        pl.kernel,
        out_type=jax.ShapeDtypeStruct((NC, NPAD, DIM), jnp.float32),
        mesh=mesh,
        scratch_types=[
            pltpu.VMEM((CPW, CHUNK), jnp.int32),
            pltpu.VMEM((GRP, CHUNK, DIM), jnp.float32),
            pltpu.VMEM_SHARED((NPAD, DIM), jnp.float32),
            pltpu.SemaphoreType.DMA,
            pltpu.SemaphoreType.DMA,
        ],
        compiler_params=params,
    )
    def sc_scatter(msg, idx, zeros, out, idx_v, rows_v, acc, sem, sem2):
        cid = lax.axis_index("c")
        sid = lax.axis_index("s")
        w = sid * NC + cid
        pltpu.sync_copy(zeros.at[pl.ds(sid * RPT, RPT)],
                        acc.at[pl.ds(sid * RPT, RPT)])
        pltpu.sync_copy(idx.at[pl.ds(w * CPW, CPW)], idx_v)
        plsc.subcore_barrier()

        def body(g, carry):
            j0 = g * GRP
            ds = [pltpu.async_copy(
                msg.at[pl.ds((w * CPW + j0 + k) * CHUNK, CHUNK), pl.ds(0, DIM)],
                rows_v.at[k], sem) for k in range(GRP)]
            for d in ds:
                d.wait()
            ds = [pltpu.async_copy(rows_v.at[k], acc.at[idx_v.at[j0 + k]],
                                   sem2, add=True) for k in range(GRP)]
            for d in ds:
                d.wait()
            return carry

        lax.fori_loop(0, CPW // GRP, body, 0)
        plsc.subcore_barrier()
        pltpu.sync_copy(acc.at[pl.ds(sid * RPT, RPT)],
                        out.at[cid, pl.ds(sid * RPT, RPT)])

    @functools.partial(
        pl.kernel,
        out_type=jax.ShapeDtypeStruct((NC, NPAD, CNTW), jnp.float32),
        mesh=mesh,
        scratch_types=[
            pltpu.VMEM((CPW, CHUNK), jnp.int32),
            pltpu.VMEM((CHUNK, CNTW), jnp.float32),
            pltpu.VMEM_SHARED((NPAD, CNTW), jnp.float32),
            pltpu.SemaphoreType.DMA,
        ],
        compiler_params=params,
    )
    def sc_count(ones, idx, zeros, out, idx_v, ones_v, acc, sem):
        cid = lax.axis_index("c")
        sid = lax.axis_index("s")
        w = sid * NC + cid
        pltpu.sync_copy(zeros.at[pl.ds(sid * RPT, RPT)],
                        acc.at[pl.ds(sid * RPT, RPT)])
        pltpu.sync_copy(idx.at[pl.ds(w * CPW, CPW)], idx_v)
        pltpu.sync_copy(ones, ones_v)
        plsc.subcore_barrier()

        def body(g, carry):
            j0 = g * GRP
            ds = [pltpu.async_copy(ones_v, acc.at[idx_v.at[j0 + k]], sem,
                                   add=True) for k in range(GRP)]
            for d in ds:
                d.wait()
            return carry

        lax.fori_loop(0, CPW // GRP, body, 0)
        plsc.subcore_barrier()
        pltpu.sync_copy(acc.at[pl.ds(sid * RPT, RPT)],
                        out.at[cid, pl.ds(sid * RPT, RPT)])

    return sc_gather, sc_scatter, sc_count


def _sc_gather(table, idx):
    return _sc_kernels()[0](table, idx)


def _sc_scatter(msg, idx, zeros):
    return _sc_kernels()[1](msg, idx, zeros)


def _sc_count(ones, idx, zeros):
    return _sc_kernels()[2](ones, idx, zeros)


def _sigmoid(v):
    return 1.0 / (1.0 + jnp.exp(-v))


def _init_body(x_r, w0_r, b0_r, out_r, outbf_r):
    o = jnp.maximum(jnp.dot(x_r[...], w0_r[...]) + b0_r[...], 0.0)
    out_r[...] = o
    outbf_r[...] = o.astype(jnp.bfloat16)


def _tc_init(x_p, w0, b0r):
    return pl.pallas_call(
        _init_body,
        out_shape=[jax.ShapeDtypeStruct((NPAD, DIM), jnp.float32),
                   jax.ShapeDtypeStruct((NPAD, DIM), jnp.bfloat16)],
    )(x_p, w0, b0r)


EBLK = 2048


def _edge_body(ea_r, xj_r, w1_r, b1_r, w2_r, b2m_r, rm_r, sm_r, msg_r):
    f32 = jnp.float32
    bf16 = jnp.bfloat16
    xj = xj_r[:, 0:DIM]
    hid = jnp.maximum(
        lax.dot_general(ea_r[...], w1_r[...], (((0,), (0,)), ((), ())))
        + b1_r[...], 0.0)
    ew = jnp.dot(hid.astype(bf16), w2_r[...], preferred_element_type=f32)
    xr = jnp.dot(xj, rm_r[...], preferred_element_type=f32)
    msg_r[:, 0:DIM] = (jnp.dot((ew * xr).astype(bf16), sm_r[...],
                               preferred_element_type=f32)
                       + jnp.dot(xj, b2m_r[...], preferred_element_type=f32))


def _tc_edge(ea_t, xj, w1, b1r, w2, b2m, rm, sm):
    zero2 = lambda i: (0, 0)
    return pl.pallas_call(
        _edge_body,
        grid=(EPAD // EBLK,),
        in_specs=[
            pl.BlockSpec((2, EBLK), lambda i: (0, i)),
            pl.BlockSpec((EBLK, 128), lambda i: (i, 0)),
            pl.BlockSpec((2, HID), zero2),
            pl.BlockSpec((1, HID), zero2),
            pl.BlockSpec((HID, DD), zero2),
            pl.BlockSpec((DIM, DIM), zero2),
            pl.BlockSpec((DIM, DD), zero2),
            pl.BlockSpec((DD, DIM), zero2),
        ],
        out_specs=pl.BlockSpec((EBLK, 128), lambda i: (i, 0)),
        out_shape=jax.ShapeDtypeStruct((EPAD, 128), jnp.float32),
    )(ea_t, xj, w1, b1r, w2, b2m, rm, sm)


def _node_body(parts_r, cnts_r, out_r, root_r, convb_r, wih_r, whh_r, bih_r,
               bhh_r, hout_r, hbf_r):
    p = parts_r[...]
    ssum = p[0] + p[1]
    c = cnts_r[...]
    cnt = c[0, :, 0:1] + c[1, :, 0:1]
    inv = 1.0 / jnp.maximum(cnt, 1.0)
    o = out_r[...]
    m = jnp.maximum(jnp.dot(o, root_r[...]) + ssum * inv + convb_r[...], 0.0)
    gi = jnp.dot(m, wih_r[...]) + bih_r[...]
    gh = jnp.dot(o, whh_r[...]) + bhh_r[...]
    r = _sigmoid(gi[:, 0:DIM] + gh[:, 0:DIM])
    z = _sigmoid(gi[:, DIM:2 * DIM] + gh[:, DIM:2 * DIM])
    n = jnp.tanh(gi[:, 2 * DIM:3 * DIM] + r * gh[:, 2 * DIM:3 * DIM])
    h = (1.0 - z) * n + z * o
    hout_r[...] = h
    hbf_r[...] = h.astype(jnp.bfloat16)


def _tc_node(parts, cnts, out, root, convbr, wih, whh, bihr, bhhr):
    return pl.pallas_call(
        _node_body,
        out_shape=[jax.ShapeDtypeStruct((NPAD, DIM), jnp.float32),
                   jax.ShapeDtypeStruct((NPAD, DIM), jnp.bfloat16)],
    )(parts, cnts, out, root, convbr, wih, whh, bihr, bhhr)


def _s2s_body(out_r, bt_r, wih_r, whh_r, bih_r, bhh_r, w1_r, b1_r, w2_r, b2_r,
              y_r):
    o = out_r[...]
    bt = bt_r[...]
    cols = lax.broadcasted_iota(jnp.int32, (NPAD, B), 1)
    oh = (bt == cols).astype(jnp.float32)
    q_star = jnp.zeros((B, 2 * DIM), jnp.float32)
    hh = jnp.zeros((B, DIM), jnp.float32)
    cc = jnp.zeros((B, DIM), jnp.float32)
    for _ in range(3):
        g = (jnp.dot(q_star, wih_r[...]) + bih_r[...]
             + jnp.dot(hh, whh_r[...]) + bhh_r[...])
        ig = _sigmoid(g[:, 0:DIM])
        fg = _sigmoid(g[:, DIM:2 * DIM])
        gg = jnp.tanh(g[:, 2 * DIM:3 * DIM])
        og = _sigmoid(g[:, 3 * DIM:4 * DIM])
        cc = fg * cc + ig * gg
        hh = og * jnp.tanh(cc)
        q = hh
        qb = jnp.dot(oh, q)
        e = jnp.sum(o * qb, axis=1, keepdims=True)
        me = jnp.where(oh > 0.0, e, -1e30)
        emax = jnp.max(me, axis=0, keepdims=True)
        emax_bc = jnp.sum(oh * emax, axis=1, keepdims=True)
        ee = jnp.exp(e - emax_bc)
        denom = jnp.sum(oh * ee, axis=0, keepdims=True)
        denom_bc = jnp.sum(oh * denom, axis=1, keepdims=True)
        a = ee / (denom_bc + 1e-16)
        r2 = lax.dot_general(oh * a, o, (((0,), (0,)), ((), ())))
        q_star = jnp.concatenate([q, r2], axis=1)
    y = jnp.maximum(jnp.dot(q_star, w1_r[...]) + b1_r[...], 0.0)
    y_r[...] = jnp.dot(y, w2_r[...]) + b2_r[...]


def _tc_s2s(out, bt_p, wih, whh, bihr, bhhr, w1, b1r, w2, b2r):
    return pl.pallas_call(
        _s2s_body,
        out_shape=jax.ShapeDtypeStruct((B, 1), jnp.float32),
    )(out, bt_p, wih, whh, bihr, bhhr, w1, b1r, w2, b2r)


def kernel(x, edge_index, edge_attr, batch, W0, b0, nnW1, nnb1, nnW2, nnb2,
           root, conv_b, gru_Wih, gru_Whh, gru_bih, gru_bhh, lstm_Wih,
           lstm_Whh, lstm_bih, lstm_bhh, W1, b1, W2, b2):
    src = edge_index[0]
    dst = edge_index[1]
    src2d = jnp.pad(src, (0, EPAD - E)).reshape(EPAD // CHUNK, CHUNK)
    dst2d = jnp.pad(dst, (0, EPAD - E), constant_values=N).reshape(
        EPAD // CHUNK, CHUNK)
    ea_t = jnp.pad(edge_attr, ((0, EPAD - E), (0, 0))).T
    x_p = jnp.pad(x, ((0, NPAD - N), (0, 0)))
    bt_p = jnp.pad(batch, (0, NPAD - N), constant_values=B).reshape(NPAD, 1)
    zeros32 = jnp.zeros((NPAD, DIM), jnp.float32)
    zeros16 = jnp.zeros((NPAD, CNTW), jnp.float32)
    ones16 = jnp.ones((CHUNK, CNTW), jnp.float32)
    rm = (jnp.arange(DD)[None, :] // DIM == jnp.arange(DIM)[:, None]
          ).astype(jnp.bfloat16)
    sm = (jnp.arange(DD)[:, None] % DIM == jnp.arange(DIM)[None, :]
          ).astype(jnp.bfloat16)
    nnW2bf = nnW2.astype(jnp.bfloat16)
    b0r = b0.reshape(1, DIM)
    nnb1r = nnb1.reshape(1, HID)
    b2m = nnb2.reshape(DIM, DIM).astype(jnp.bfloat16)
    convbr = conv_b.reshape(1, DIM)
    bihr = gru_bih.reshape(1, 3 * DIM)
    bhhr = gru_bhh.reshape(1, 3 * DIM)
    lbih = lstm_bih.reshape(1, 4 * DIM)
    lbhh = lstm_bhh.reshape(1, 4 * DIM)
    b1r = b1.reshape(1, DIM)
    b2r = b2.reshape(1, 1)

    out, table = _tc_init(x_p, W0, b0r)
    cnts = _sc_count(ones16, dst2d, zeros16)
    for _ in range(3):
        xj = _sc_gather(table, src2d)
        msg = _tc_edge(ea_t, xj, nnW1, nnb1r, nnW2bf, b2m, rm, sm)
        parts = _sc_scatter(msg, dst2d, zeros32)
        out, table = _tc_node(parts, cnts, out, root, convbr, gru_Wih,
                              gru_Whh, bihr, bhhr)
    y = _tc_s2s(out, bt_p, lstm_Wih, lstm_Whh, lbih, lbhh, W1, b1r, W2, b2r)
    return y.reshape(-1)


# speedup vs baseline: 1.3814x; 1.3814x over previous
import functools

import jax
import jax.numpy as jnp
from jax import lax
from jax.experimental import pallas as pl
from jax.experimental.pallas import tpu as pltpu
from jax.experimental.pallas import tpu_sc as plsc

N = 10000
E = 160000
B = 64
DIM = 32
HID = 128
DD = DIM * DIM

NC = 2
NS = 16
NW = NC * NS
CHUNK = 128
NPAD = 10016
EPAD = 163840
CPW = EPAD // (NW * CHUNK)
RPT = NPAD // NS
EH = EPAD // 2
CPWH = EH // (NW * CHUNK)
CNTW = 16
GRP = 8


@functools.cache
def _sc_kernels():
    mesh = plsc.VectorSubcoreMesh(core_axis_name="c", subcore_axis_name="s",
                                  num_cores=NC, num_subcores=NS)
    params = pltpu.CompilerParams(use_tc_tiling_on_sc=False)

    def make_gather(nedge, cpw):
        @functools.partial(
            pl.kernel,
            out_type=jax.ShapeDtypeStruct((nedge, 128), jnp.float32),
            mesh=mesh,
            scratch_types=[
                pltpu.VMEM((cpw, CHUNK), jnp.int32),
                pltpu.VMEM((GRP, CHUNK, DIM), jnp.float32),
                pltpu.SemaphoreType.DMA,
                pltpu.SemaphoreType.DMA,
            ],
            compiler_params=params,
        )
        def sc_gather(table, idx, out, idx_v, rows_v, sem, sem2):
            w = lax.axis_index("s") * NC + lax.axis_index("c")
            base = w * cpw
            pltpu.sync_copy(idx.at[pl.ds(base, cpw)], idx_v)

            def body(g, carry):
                j0 = g * GRP
                ds = [pltpu.async_copy(table.at[idx_v.at[j0 + k]],
                                       rows_v.at[k], sem) for k in range(GRP)]
                for d in ds:
                    d.wait()
                ds = [pltpu.async_copy(
                    rows_v.at[k],
                    out.at[pl.ds((base + j0 + k) * CHUNK, CHUNK),
                           pl.ds(0, DIM)], sem2) for k in range(GRP)]
                for d in ds:
                    d.wait()
                return carry

            lax.fori_loop(0, cpw // GRP, body, 0)

        return sc_gather

    def make_scatter(cpw):
        @functools.partial(
            pl.kernel,
            out_type=jax.ShapeDtypeStruct((NC, NPAD, DIM), jnp.float32),
            mesh=mesh,
            scratch_types=[
                pltpu.VMEM((cpw, CHUNK), jnp.int32),
                pltpu.VMEM((GRP, CHUNK, DIM), jnp.float32),
                pltpu.VMEM_SHARED((NPAD, DIM), jnp.float32),
                pltpu.SemaphoreType.DMA,
                pltpu.SemaphoreType.DMA,
            ],
            compiler_params=params,
        )
        def sc_scatter(msg, idx, zeros, out, idx_v, rows_v, acc, sem, sem2):
            cid = lax.axis_index("c")
            sid = lax.axis_index("s")
            w = sid * NC + cid
            pltpu.sync_copy(zeros.at[pl.ds(sid * RPT, RPT)],
                            acc.at[pl.ds(sid * RPT, RPT)])
            pltpu.sync_copy(idx.at[pl.ds(w * cpw, cpw)], idx_v)
            plsc.subcore_barrier()

            def body(g, carry):
                j0 = g * GRP
                ds = [pltpu.async_copy(
                    msg.at[pl.ds((w * cpw + j0 + k) * CHUNK, CHUNK),
                           pl.ds(0, DIM)], rows_v.at[k], sem)
                    for k in range(GRP)]
                for d in ds:
                    d.wait()
                ds = [pltpu.async_copy(rows_v.at[k], acc.at[idx_v.at[j0 + k]],
                                       sem2, add=True) for k in range(GRP)]
                for d in ds:
                    d.wait()
                return carry

            lax.fori_loop(0, cpw // GRP, body, 0)
            plsc.subcore_barrier()
            pltpu.sync_copy(acc.at[pl.ds(sid * RPT, RPT)],
                            out.at[cid, pl.ds(sid * RPT, RPT)])

        return sc_scatter

    sc_gather = make_gather(EH, CPWH)
    sc_scatter = make_scatter(CPWH)

    @functools.partial(
        pl.kernel,
        out_type=jax.ShapeDtypeStruct((NC, NPAD, CNTW), jnp.float32),
        mesh=mesh,
        scratch_types=[
            pltpu.VMEM((CPW, CHUNK), jnp.int32),
            pltpu.VMEM((CHUNK, CNTW), jnp.float32),
            pltpu.VMEM_SHARED((NPAD, CNTW), jnp.float32),
            pltpu.SemaphoreType.DMA,
        ],
        compiler_params=params,
    )
    def sc_count(ones, idx, zeros, out, idx_v, ones_v, acc, sem):
        cid = lax.axis_index("c")
        sid = lax.axis_index("s")
        w = sid * NC + cid
        pltpu.sync_copy(zeros.at[pl.ds(sid * RPT, RPT)],
                        acc.at[pl.ds(sid * RPT, RPT)])
        pltpu.sync_copy(idx.at[pl.ds(w * CPW, CPW)], idx_v)
        pltpu.sync_copy(ones, ones_v)
        plsc.subcore_barrier()

        def body(g, carry):
            j0 = g * GRP
            ds = [pltpu.async_copy(ones_v, acc.at[idx_v.at[j0 + k]], sem,
                                   add=True) for k in range(GRP)]
            for d in ds:
                d.wait()
            return carry

        lax.fori_loop(0, CPW // GRP, body, 0)
        plsc.subcore_barrier()
        pltpu.sync_copy(acc.at[pl.ds(sid * RPT, RPT)],
                        out.at[cid, pl.ds(sid * RPT, RPT)])

    return sc_gather, sc_scatter, sc_count


def _sc_gather(table, idx):
    return _sc_kernels()[0](table, idx)


def _sc_scatter(msg, idx, zeros):
    return _sc_kernels()[1](msg, idx, zeros)


def _sc_count(ones, idx, zeros):
    return _sc_kernels()[2](ones, idx, zeros)


def _sigmoid(v):
    return 1.0 / (1.0 + jnp.exp(-v))


def _init_body(x_r, w0_r, b0_r, out_r):
    out_r[...] = jnp.maximum(jnp.dot(x_r[...], w0_r[...]) + b0_r[...], 0.0)


def _tc_init(x_p, w0, b0r):
    return pl.pallas_call(
        _init_body,
        out_shape=jax.ShapeDtypeStruct((NPAD, DIM), jnp.float32),
    )(x_p, w0, b0r)


EBLK = 2048


def _edge_body(ea_r, xj_r, w1_r, b1_r, w2_r, b2m_r, rm_r, sm_r, msg_r):
    f32 = jnp.float32
    bf16 = jnp.bfloat16
    xj = xj_r[:, 0:DIM]
    hid = jnp.maximum(
        lax.dot_general(ea_r[...], w1_r[...], (((0,), (0,)), ((), ())))
        + b1_r[...], 0.0)
    ew = jnp.dot(hid.astype(bf16), w2_r[...], preferred_element_type=f32)
    xr = jnp.dot(xj.astype(bf16), rm_r[...], preferred_element_type=f32)
    msg_r[:, 0:DIM] = (jnp.dot((ew * xr).astype(bf16), sm_r[...],
                               preferred_element_type=f32)
                       + jnp.dot(xj, b2m_r[...]))


def _tc_edge(ea_t, xj, w1, b1r, w2, b2m, rm, sm):
    zero2 = lambda i: (0, 0)
    return pl.pallas_call(
        _edge_body,
        grid=(EH // EBLK,),
        in_specs=[
            pl.BlockSpec((2, EBLK), lambda i: (0, i)),
            pl.BlockSpec((EBLK, 128), lambda i: (i, 0)),
            pl.BlockSpec((2, HID), zero2),
            pl.BlockSpec((1, HID), zero2),
            pl.BlockSpec((HID, DD), zero2),
            pl.BlockSpec((DIM, DIM), zero2),
            pl.BlockSpec((DIM, DD), zero2),
            pl.BlockSpec((DD, DIM), zero2),
        ],
        out_specs=pl.BlockSpec((EBLK, 128), lambda i: (i, 0)),
        out_shape=jax.ShapeDtypeStruct((EH, 128), jnp.float32),
    )(ea_t, xj, w1, b1r, w2, b2m, rm, sm)


def _node_body(pa_r, pb_r, cnts_r, out_r, root_r, convb_r, wih_r, whh_r,
               bih_r, bhh_r, hout_r):
    pa = pa_r[...]
    pb = pb_r[...]
    ssum = pa[0] + pa[1] + pb[0] + pb[1]
    c = cnts_r[...]
    cnt = c[0, :, 0:1] + c[1, :, 0:1]
    inv = 1.0 / jnp.maximum(cnt, 1.0)
    o = out_r[...]
    m = jnp.maximum(jnp.dot(o, root_r[...]) + ssum * inv + convb_r[...], 0.0)
    gi = jnp.dot(m, wih_r[...]) + bih_r[...]
    gh = jnp.dot(o, whh_r[...]) + bhh_r[...]
    r = _sigmoid(gi[:, 0:DIM] + gh[:, 0:DIM])
    z = _sigmoid(gi[:, DIM:2 * DIM] + gh[:, DIM:2 * DIM])
    n = jnp.tanh(gi[:, 2 * DIM:3 * DIM] + r * gh[:, 2 * DIM:3 * DIM])
    hout_r[...] = (1.0 - z) * n + z * o


def _tc_node(pa, pb, cnts, out, root, convbr, wih, whh, bihr, bhhr):
    return pl.pallas_call(
        _node_body,
        out_shape=jax.ShapeDtypeStruct((NPAD, DIM), jnp.float32),
    )(pa, pb, cnts, out, root, convbr, wih, whh, bihr, bhhr)


def _s2s_body(out_r, bt_r, wih_r, whh_r, bih_r, bhh_r, w1_r, b1_r, w2_r, b2_r,
              y_r):
    o = out_r[...]
    bt = bt_r[...]
    cols = lax.broadcasted_iota(jnp.int32, (NPAD, B), 1)
    oh = (bt == cols).astype(jnp.float32)
    q_star = jnp.zeros((B, 2 * DIM), jnp.float32)
    hh = jnp.zeros((B, DIM), jnp.float32)
    cc = jnp.zeros((B, DIM), jnp.float32)
    for _ in range(3):
        g = (jnp.dot(q_star, wih_r[...]) + bih_r[...]
             + jnp.dot(hh, whh_r[...]) + bhh_r[...])
        ig = _sigmoid(g[:, 0:DIM])
        fg = _sigmoid(g[:, DIM:2 * DIM])
        gg = jnp.tanh(g[:, 2 * DIM:3 * DIM])
        og = _sigmoid(g[:, 3 * DIM:4 * DIM])
        cc = fg * cc + ig * gg
        hh = og * jnp.tanh(cc)
        q = hh
        qb = jnp.dot(oh, q)
        e = jnp.sum(o * qb, axis=1, keepdims=True)
        me = jnp.where(oh > 0.0, e, -1e30)
        emax = jnp.max(me, axis=0, keepdims=True)
        emax_bc = jnp.sum(oh * emax, axis=1, keepdims=True)
        ee = jnp.exp(e - emax_bc)
        denom = jnp.sum(oh * ee, axis=0, keepdims=True)
        denom_bc = jnp.sum(oh * denom, axis=1, keepdims=True)
        a = ee / (denom_bc + 1e-16)
        r2 = lax.dot_general(oh * a, o, (((0,), (0,)), ((), ())))
        q_star = jnp.concatenate([q, r2], axis=1)
    y = jnp.maximum(jnp.dot(q_star, w1_r[...]) + b1_r[...], 0.0)
    y_r[...] = jnp.dot(y, w2_r[...]) + b2_r[...]


def _tc_s2s(out, bt_p, wih, whh, bihr, bhhr, w1, b1r, w2, b2r):
    return pl.pallas_call(
        _s2s_body,
        out_shape=jax.ShapeDtypeStruct((B, 1), jnp.float32),
    )(out, bt_p, wih, whh, bihr, bhhr, w1, b1r, w2, b2r)


def kernel(x, edge_index, edge_attr, batch, W0, b0, nnW1, nnb1, nnW2, nnb2,
           root, conv_b, gru_Wih, gru_Whh, gru_bih, gru_bhh, lstm_Wih,
           lstm_Whh, lstm_bih, lstm_bhh, W1, b1, W2, b2):
    src = edge_index[0]
    dst = edge_index[1]
    src2d = jnp.pad(src, (0, EPAD - E)).reshape(EPAD // CHUNK, CHUNK)
    dst2d = jnp.pad(dst, (0, EPAD - E), constant_values=N).reshape(
        EPAD // CHUNK, CHUNK)
    ea_t = jnp.pad(edge_attr, ((0, EPAD - E), (0, 0))).T
    x_p = jnp.pad(x, ((0, NPAD - N), (0, 0)))
    bt_p = jnp.pad(batch, (0, NPAD - N), constant_values=B).reshape(NPAD, 1)
    zeros32 = jnp.zeros((NPAD, DIM), jnp.float32)
    zeros16 = jnp.zeros((NPAD, CNTW), jnp.float32)
    ones16 = jnp.ones((CHUNK, CNTW), jnp.float32)
    rm = (jnp.arange(DD)[None, :] // DIM == jnp.arange(DIM)[:, None]
          ).astype(jnp.bfloat16)
    sm = (jnp.arange(DD)[:, None] % DIM == jnp.arange(DIM)[None, :]
          ).astype(jnp.bfloat16)
    nnW2bf = nnW2.astype(jnp.bfloat16)
    b0r = b0.reshape(1, DIM)
    nnb1r = nnb1.reshape(1, HID)
    b2m = nnb2.reshape(DIM, DIM)
    convbr = conv_b.reshape(1, DIM)
    bihr = gru_bih.reshape(1, 3 * DIM)
    bhhr = gru_bhh.reshape(1, 3 * DIM)
    lbih = lstm_bih.reshape(1, 4 * DIM)
    lbhh = lstm_bhh.reshape(1, 4 * DIM)
    b1r = b1.reshape(1, DIM)
    b2r = b2.reshape(1, 1)

    ea_ta = ea_t[:, :EH]
    ea_tb = ea_t[:, EH:]
    nh = EH // CHUNK
    src_a, src_b = src2d[:nh], src2d[nh:]
    dst_a, dst_b = dst2d[:nh], dst2d[nh:]

    out = _tc_init(x_p, W0, b0r)
    cnts = _sc_count(ones16, dst2d, zeros16)
    for _ in range(3):
        xj_a = _sc_gather(out, src_a)
        xj_b = _sc_gather(out, src_b)
        msg_a = _tc_edge(ea_ta, xj_a, nnW1, nnb1r, nnW2bf, b2m, rm, sm)
        msg_b = _tc_edge(ea_tb, xj_b, nnW1, nnb1r, nnW2bf, b2m, rm, sm)
        pa = _sc_scatter(msg_a, dst_a, zeros32)
        pb = _sc_scatter(msg_b, dst_b, zeros32)
        out = _tc_node(pa, pb, cnts, out, root, convbr, gru_Wih, gru_Whh,
                       bihr, bhhr)
    y = _tc_s2s(out, bt_p, lstm_Wih, lstm_Whh, lbih, lbhh, W1, b1r, W2, b2r)
    return y.reshape(-1)
